# Initial kernel scaffold; baseline (speedup 1.0000x reference)
import jax
import jax.numpy as jnp
from jax.experimental import pallas as pl


def kernel(output_13, output_26, output_52, anchors, threshold):
    raise NotImplementedError("write your pallas kernel here")


# speedup vs baseline: 4.5821x; 4.5821x over previous
import functools

import jax
import jax.numpy as jnp
from jax.experimental import pallas as pl
from jax.experimental.pallas import tpu as pltpu


def _body(params_ref, cells_ref, fill_ref, out_ref, sm, carry_ref,
          *, nb, n_valid, H, W, t):
    j = pl.program_id(0)
    thr = params_ref[0]
    aw0 = params_ref[1]
    aw1 = params_ref[2]
    aw2 = params_ref[3]
    ah = params_ref[4]

    fill_blk = fill_ref[...]

    @pl.when(j == 0)
    def _init():
        sm[0] = 0
        sm[1] = 0
        out_ref[...] = jnp.broadcast_to(fill_blk[None], (nb, 6, 128))

    block = cells_ref[0]
    conf = block[0:1, :]
    tx = block[1:2, :]
    ty = block[2:3, :]
    tw = block[3:4, :]
    th = block[4:5, :]
    cls_scores = block[5:85, :]

    sub80 = jax.lax.broadcasted_iota(jnp.int32, (80, 128), 0)
    cmax = jnp.max(cls_scores, axis=0, keepdims=True)
    cls_idx = jnp.min(jnp.where(cls_scores == cmax, sub80, 10000), axis=0)
    cls = cls_idx.astype(jnp.float32).reshape(1, 128)

    lane = jax.lax.broadcasted_iota(jnp.int32, (1, 128), 1)
    lin = j * 128 + lane
    a = lin % 3
    cell = lin // 3
    x = cell % W
    y = (cell // W) % H

    aw = jnp.where(a == 0, aw0, jnp.where(a == 1, aw1, aw2))
    cy = (y.astype(jnp.float32) + ty) * t
    cx = (x.astype(jnp.float32) + tx) * t
    w = aw * jnp.exp(tw)
    h = ah * jnp.exp(th)
    x1 = cx - w * 0.5
    y1 = cy - h * 0.5
    rows6 = jnp.concatenate([conf, x1, y1, x1 + w, y1 + h, cls], axis=0)

    mask = (conf > thr) & (lin < n_valid)
    mf = mask.astype(jnp.float32)
    li = jax.lax.broadcasted_iota(jnp.int32, (128, 128), 1)
    si = jax.lax.broadcasted_iota(jnp.int32, (128, 128), 0)
    lt = (si <= li).astype(jnp.float32)
    csum = jax.lax.dot_general(
        mf, lt, (((1,), (0,)), ((), ())),
        preferred_element_type=jnp.float32,
        precision=jax.lax.Precision.HIGHEST)
    cnt = csum[0, 127].astype(jnp.int32)
    r = csum.astype(jnp.int32) - 1

    c = sm[0]
    chunk = sm[1]
    tgt = (r + c) % 128

    tgt_b = jnp.broadcast_to(tgt, (128, 128))
    mask_b = jnp.broadcast_to(mask, (128, 128))
    pmt = ((tgt_b == si) & mask_b).astype(jnp.float32)
    merged = jax.lax.dot_general(
        rows6, pmt, (((1,), (1,)), ((), ())),
        preferred_element_type=jnp.float32,
        precision=jax.lax.Precision.HIGHEST)

    carry_old = carry_ref[...]
    flush = (c + cnt) >= 128

    @pl.when(flush)
    def _flush():
        fv = jnp.where(lane >= c, merged, carry_old)
        out_ref[chunk] = fv
        carry_ref[...] = merged
        sm[0] = c + cnt - 128
        sm[1] = chunk + 1

    @pl.when(jnp.logical_not(flush))
    def _accum():
        new_lane = ((lane - c) % 128) < cnt
        carry_ref[...] = jnp.where(new_lane, merged, carry_old)
        sm[0] = c + cnt

    @pl.when(j == nb - 1)
    def _tail():
        c2 = sm[0]
        ch2 = sm[1]

        @pl.when(ch2 < nb)
        def _store_tail():
            fin = jnp.where(lane < c2, carry_ref[...], fill_blk)
            out_ref[ch2] = fin


def _compact_scale(out, thr, anchors_s, t):
    B, C, H, W = out.shape
    N = B * H * W * 3
    nb = -(-N // 128)
    Np = nb * 128
    cells = out.reshape(B, 3, 85, H, W).transpose(2, 0, 3, 4, 1).reshape(85, N)
    cells = jnp.pad(cells, ((0, 0), (0, Np - N)))
    cells = cells.reshape(85, nb, 128).transpose(1, 0, 2)

    f0 = out[0, :85, 0, 0]
    cls0 = jnp.argmax(f0[5:]).astype(jnp.float32)
    cy0 = f0[2] * t
    cx0 = f0[1] * t
    w0 = anchors_s[0, 0] * jnp.exp(f0[3])
    h0 = anchors_s[1, 1] * jnp.exp(f0[4])
    x10 = cx0 - w0 * 0.5
    y10 = cy0 - h0 * 0.5
    fill = jnp.stack([f0[0], x10, y10, x10 + w0, y10 + h0, cls0])
    fill_arr = jnp.broadcast_to(fill.reshape(6, 1), (6, 128))

    params = jnp.stack([thr, anchors_s[0, 0], anchors_s[1, 0],
                        anchors_s[2, 0], anchors_s[1, 1],
                        jnp.float32(0.0), jnp.float32(0.0), jnp.float32(0.0)])

    body = functools.partial(_body, nb=nb, n_valid=N, H=H, W=W, t=t)
    res = pl.pallas_call(
        body,
        grid=(nb,),
        in_specs=[
            pl.BlockSpec(memory_space=pltpu.SMEM),
            pl.BlockSpec((1, 85, 128), lambda j: (j, 0, 0)),
            pl.BlockSpec((6, 128), lambda j: (0, 0)),
        ],
        out_specs=pl.BlockSpec((nb, 6, 128), lambda j: (0, 0, 0)),
        out_shape=jax.ShapeDtypeStruct((nb, 6, 128), jnp.float32),
        scratch_shapes=[pltpu.SMEM((2,), jnp.int32),
                        pltpu.VMEM((6, 128), jnp.float32)],
    )(params, cells, fill_arr)
    return res.transpose(0, 2, 1).reshape(Np, 6)[:N]


def _kernel_impl(output_13, output_26, output_52, anchors, threshold):
    thr = threshold.reshape(())
    b13 = _compact_scale(output_13, thr, anchors[0], 32.0)
    b26 = _compact_scale(output_26, thr, anchors[1], 16.0)
    b52 = _compact_scale(output_52, thr, anchors[2], 8.0)
    return jnp.concatenate([b13, b26, b52], axis=0)


kernel = jax.jit(_kernel_impl)


# speedup vs baseline: 5.2606x; 1.1481x over previous
import functools

import jax
import jax.numpy as jnp
from jax.experimental import pallas as pl
from jax.experimental.pallas import tpu as pltpu

_U = 4


def _body(params_ref, cells_ref, fill_ref, out_ref, sm, carry_ref,
          *, ng, nb4, n_valid, H, W, t):
    j = pl.program_id(0)
    thr = params_ref[0]
    aw0 = params_ref[1]
    aw1 = params_ref[2]
    aw2 = params_ref[3]
    ah = params_ref[4]

    fill_blk = fill_ref[...]
    lane = jax.lax.broadcasted_iota(jnp.int32, (1, 128), 1)
    sub80 = jax.lax.broadcasted_iota(jnp.int32, (80, 128), 0)
    li = jax.lax.broadcasted_iota(jnp.int32, (128, 128), 1)
    si = jax.lax.broadcasted_iota(jnp.int32, (128, 128), 0)
    lt = (si <= li).astype(jnp.float32)

    @pl.when(j == 0)
    def _init():
        sm[0] = 0
        sm[1] = 0
        out_ref[...] = jnp.broadcast_to(fill_blk[None], (nb4, 6, 128))

    for u in range(_U):
        block = cells_ref[u]
        conf = block[0:1, :]
        tx = block[1:2, :]
        ty = block[2:3, :]
        tw = block[3:4, :]
        th = block[4:5, :]
        cls_scores = block[5:85, :]

        cmax = jnp.max(cls_scores, axis=0, keepdims=True)
        cls_idx = jnp.min(jnp.where(cls_scores == cmax, sub80, 10000), axis=0)
        cls = cls_idx.astype(jnp.float32).reshape(1, 128)

        lin = (j * _U + u) * 128 + lane
        a = lin % 3
        cell = lin // 3
        x = cell % W
        y = (cell // W) % H

        aw = jnp.where(a == 0, aw0, jnp.where(a == 1, aw1, aw2))
        cy = (y.astype(jnp.float32) + ty) * t
        cx = (x.astype(jnp.float32) + tx) * t
        w = aw * jnp.exp(tw)
        h = ah * jnp.exp(th)
        x1 = cx - w * 0.5
        y1 = cy - h * 0.5
        rows6 = jnp.concatenate([conf, x1, y1, x1 + w, y1 + h, cls], axis=0)

        mask = (conf > thr) & (lin < n_valid)
        mf = mask.astype(jnp.float32)
        csum = jax.lax.dot_general(
            mf, lt, (((1,), (0,)), ((), ())),
            preferred_element_type=jnp.float32,
            precision=jax.lax.Precision.HIGHEST)
        cnt = csum[0, 127].astype(jnp.int32)
        r = csum.astype(jnp.int32) - 1

        c = sm[0]
        chunk = sm[1]
        tgt = (r + c) % 128

        tgt_b = jnp.broadcast_to(tgt, (128, 128))
        mask_b = jnp.broadcast_to(mask, (128, 128))
        pmt = ((tgt_b == si) & mask_b).astype(jnp.float32)
        merged = jax.lax.dot_general(
            rows6, pmt, (((1,), (1,)), ((), ())),
            preferred_element_type=jnp.float32,
            precision=jax.lax.Precision.HIGHEST)

        carry_old = carry_ref[...]
        flush = (c + cnt) >= 128

        @pl.when(flush)
        def _flush():
            fv = jnp.where(lane >= c, merged, carry_old)
            out_ref[chunk] = fv
            carry_ref[...] = merged
            sm[0] = c + cnt - 128
            sm[1] = chunk + 1

        @pl.when(jnp.logical_not(flush))
        def _accum():
            new_lane = ((lane - c) % 128) < cnt
            carry_ref[...] = jnp.where(new_lane, merged, carry_old)
            sm[0] = c + cnt

    @pl.when(j == ng - 1)
    def _tail():
        c2 = sm[0]
        ch2 = sm[1]

        @pl.when(ch2 < nb4)
        def _store_tail():
            fin = jnp.where(lane < c2, carry_ref[...], fill_blk)
            out_ref[ch2] = fin


def _compact_scale(out, thr, anchors_s, t):
    B, C, H, W = out.shape
    N = B * H * W * 3
    nb = -(-N // 128)
    ng = -(-nb // _U)
    nb4 = ng * _U
    Np = nb4 * 128
    cells = out.reshape(B, 3, 85, H, W).transpose(2, 0, 3, 4, 1).reshape(85, N)
    cells = jnp.pad(cells, ((0, 0), (0, Np - N)))
    cells = cells.reshape(85, nb4, 128).transpose(1, 0, 2)

    f0 = out[0, :85, 0, 0]
    cls0 = jnp.argmax(f0[5:]).astype(jnp.float32)
    cy0 = f0[2] * t
    cx0 = f0[1] * t
    w0 = anchors_s[0, 0] * jnp.exp(f0[3])
    h0 = anchors_s[1, 1] * jnp.exp(f0[4])
    x10 = cx0 - w0 * 0.5
    y10 = cy0 - h0 * 0.5
    fill = jnp.stack([f0[0], x10, y10, x10 + w0, y10 + h0, cls0])
    fill_arr = jnp.broadcast_to(fill.reshape(6, 1), (6, 128))

    params = jnp.stack([thr, anchors_s[0, 0], anchors_s[1, 0],
                        anchors_s[2, 0], anchors_s[1, 1],
                        jnp.float32(0.0), jnp.float32(0.0), jnp.float32(0.0)])

    body = functools.partial(_body, ng=ng, nb4=nb4, n_valid=N, H=H, W=W, t=t)
    res = pl.pallas_call(
        body,
        grid=(ng,),
        in_specs=[
            pl.BlockSpec(memory_space=pltpu.SMEM),
            pl.BlockSpec((_U, 85, 128), lambda j: (j, 0, 0)),
            pl.BlockSpec((6, 128), lambda j: (0, 0)),
        ],
        out_specs=pl.BlockSpec((nb4, 6, 128), lambda j: (0, 0, 0)),
        out_shape=jax.ShapeDtypeStruct((nb4, 6, 128), jnp.float32),
        scratch_shapes=[pltpu.SMEM((2,), jnp.int32),
                        pltpu.VMEM((6, 128), jnp.float32)],
    )(params, cells, fill_arr)
    return res.transpose(0, 2, 1).reshape(Np, 6)[:N]


def _kernel_impl(output_13, output_26, output_52, anchors, threshold):
    thr = threshold.reshape(())
    b13 = _compact_scale(output_13, thr, anchors[0], 32.0)
    b26 = _compact_scale(output_26, thr, anchors[1], 16.0)
    b52 = _compact_scale(output_52, thr, anchors[2], 8.0)
    return jnp.concatenate([b13, b26, b52], axis=0)


kernel = jax.jit(_kernel_impl)
